# baseline (speedup 1.0000x reference)
import functools

import jax
import jax.numpy as jnp
from jax import lax
from jax.experimental import pallas as pl
from jax.experimental.pallas import tpu as pltpu
from jax.experimental.pallas import tpu_sc as plsc

EPS = 1e-3
WEIGHT_START = 5000.0
WEIGHT_MAX = 500000.0
START_RAMPUP = 50000
N_RAMPUP = 100000

N_OBS = 6890
N_FACES = 13776
N_CLOTH = 20000

F_PAD = 13824
F_PER_TILE = 432
F_GROUPS = F_PER_TILE // 16

Q_PAD = 20480
Q_PER_TILE = 640
Q_GROUPS = Q_PER_TILE // 16

NC, NS, L = 2, 16, 16
NW = NC * NS


def _splat_i32(v):
    return jnp.full((L,), v, dtype=jnp.int32)


def _rsqrt_newton(s):
    bits = plsc.bitcast(s, jnp.int32)
    r = plsc.bitcast(jnp.int32(0x5F3759DF) - (bits >> 1), jnp.float32)
    for _ in range(3):
        r = r * (1.5 - 0.5 * s * r * r)
    return r


def _sc_prep_body(pos_hbm, prev_hbm, faces_hbm, *refs):
    out_hbm = refs[:10]
    pos_v, prev_v, faces_v = refs[10:13]
    buf_v = refs[13:23]
    wid = lax.axis_index("s") * NC + lax.axis_index("c")
    base = wid * F_PER_TILE
    pltpu.sync_copy(pos_hbm, pos_v)
    pltpu.sync_copy(prev_hbm, prev_v)
    pltpu.sync_copy(faces_hbm.at[pl.ds(base * 3, F_PER_TILE * 3)], faces_v)

    iota = lax.iota(jnp.int32, L)

    def body(g, carry):
        ridx = g * L + iota
        fidx = ridx * 3
        i0 = plsc.load_gather(faces_v, [fidx]) * 3
        i1 = plsc.load_gather(faces_v, [fidx + 1]) * 3
        i2 = plsc.load_gather(faces_v, [fidx + 2]) * 3
        valid = (base + ridx) < N_FACES
        cur = []
        keyc = []
        for c in range(3):
            p0 = plsc.load_gather(prev_v, [i0 + c])
            p1 = plsc.load_gather(prev_v, [i1 + c])
            p2 = plsc.load_gather(prev_v, [i2 + c])
            key = (p0 + p1 + p2) / 3.0
            key = jnp.where(valid, key, 1e9)
            keyc.append(key)
            plsc.store_scatter(buf_v[c], [ridx], key)
            v0 = plsc.load_gather(pos_v, [i0 + c])
            v1 = plsc.load_gather(pos_v, [i1 + c])
            v2 = plsc.load_gather(pos_v, [i2 + c])
            plsc.store_scatter(buf_v[3 + c], [ridx], (v0 + v1 + v2) / 3.0)
            cur.append((v0, v1, v2))
        khalf = (keyc[0] * keyc[0] + keyc[1] * keyc[1]
                 + keyc[2] * keyc[2]) * 0.5
        plsc.store_scatter(buf_v[9], [ridx], khalf)
        (v0x, v1x, v2x), (v0y, v1y, v2y), (v0z, v1z, v2z) = cur
        e1x, e1y, e1z = v1x - v0x, v1y - v0y, v1z - v0z
        e2x, e2y, e2z = v2x - v0x, v2y - v0y, v2z - v0z
        crx = e1y * e2z - e1z * e2y
        cry = e1z * e2x - e1x * e2z
        crz = e1x * e2y - e1y * e2x
        s = crx * crx + cry * cry + crz * crz
        r = _rsqrt_newton(s)
        inv = 1.0 / (s * r + 1e-8)
        plsc.store_scatter(buf_v[6], [ridx], crx * inv)
        plsc.store_scatter(buf_v[7], [ridx], cry * inv)
        plsc.store_scatter(buf_v[8], [ridx], crz * inv)
        return carry

    lax.fori_loop(0, F_GROUPS, body, 0)
    for r in range(10):
        pltpu.sync_copy(buf_v[r], out_hbm[r].at[pl.ds(base, F_PER_TILE)])


@functools.lru_cache(maxsize=None)
def _sc_prep_kernel():
    return pl.kernel(
        _sc_prep_body,
        out_type=[jax.ShapeDtypeStruct((F_PAD,), jnp.float32)] * 10,
        mesh=plsc.VectorSubcoreMesh(core_axis_name="c", subcore_axis_name="s"),
        compiler_params=pltpu.CompilerParams(needs_layout_passes=False),
        scratch_types=[
            pltpu.VMEM((N_OBS * 3,), jnp.float32),
            pltpu.VMEM((N_OBS * 3,), jnp.float32),
            pltpu.VMEM((F_PER_TILE * 3,), jnp.int32),
        ] + [pltpu.VMEM((F_PER_TILE,), jnp.float32)] * 10,
    )


def _tc_knn_body(q_ref, kx_ref, ky_ref, kz_ref, kh_ref, out_ref):
    NK = F_PAD // 128
    for j in range(16):
        q = q_ref[pl.ds(j * 64, 64), :]
        qx = q[:, 0:1]
        qy = q[:, 1:2]
        qz = q[:, 2:3]
        m = jnp.full((64, 128), jnp.inf, dtype=jnp.float32)
        idx = jnp.zeros((64, 128), dtype=jnp.int32)

        def chunk(kc, carry):
            m, idx = carry
            sl = pl.ds(kc * 128, 128)
            t = qx * kx_ref[0:1, sl]
            t = t + qy * ky_ref[0:1, sl]
            t = t + qz * kz_ref[0:1, sl]
            score = kh_ref[0:1, sl] - t
            better = score < m
            m = jnp.where(better, score, m)
            idx = jnp.where(better, kc, idx)
            return m, idx

        m, idx = lax.fori_loop(0, NK, chunk, (m, idx), unroll=4)
        lane = lax.broadcasted_iota(jnp.int32, (64, 128), 1)
        glob = idx * 128 + lane
        dmin = jnp.min(m, axis=1, keepdims=True)
        cand = jnp.where(m == dmin, glob, 1 << 30)
        res = jnp.min(cand, axis=1)
        out_ref[0, j // 2, pl.ds((j % 2) * 64, 64)] = res


_tc_knn = pl.pallas_call(
    _tc_knn_body,
    grid=(Q_PAD // 1024,),
    in_specs=[
        pl.BlockSpec((1024, 3), lambda i: (i, 0)),
        pl.BlockSpec((1, F_PAD), lambda i: (0, 0)),
        pl.BlockSpec((1, F_PAD), lambda i: (0, 0)),
        pl.BlockSpec((1, F_PAD), lambda i: (0, 0)),
        pl.BlockSpec((1, F_PAD), lambda i: (0, 0)),
    ],
    out_specs=pl.BlockSpec((1, 8, 128), lambda i: (i, 0, 0)),
    out_shape=jax.ShapeDtypeStruct((Q_PAD // 1024, 8, 128), jnp.int32),
)


def _sc_pen_body(cx_hbm, cy_hbm, cz_hbm, nx_hbm, ny_hbm, nz_hbm,
                 nn_hbm, pred_hbm, out_hbm,
                 cx_v, cy_v, cz_v, nx_v, ny_v, nz_v, nn_v, pred_v, acc_v):
    wid = lax.axis_index("s") * NC + lax.axis_index("c")
    base = wid * Q_PER_TILE
    for h, v in ((cx_hbm, cx_v), (cy_hbm, cy_v), (cz_hbm, cz_v),
                 (nx_hbm, nx_v), (ny_hbm, ny_v), (nz_hbm, nz_v)):
        pltpu.sync_copy(h, v)
    pltpu.sync_copy(nn_hbm.at[pl.ds(base, Q_PER_TILE)], nn_v)
    pltpu.sync_copy(pred_hbm.at[pl.ds(base * 3, Q_PER_TILE * 3)], pred_v)

    iota = lax.iota(jnp.int32, L)

    def body(g, acc):
        ridx = g * L + iota
        qidx = ridx * 3
        nn = plsc.load_gather(nn_v, [ridx])
        qx = plsc.load_gather(pred_v, [qidx])
        qy = plsc.load_gather(pred_v, [qidx + 1])
        qz = plsc.load_gather(pred_v, [qidx + 2])
        px = plsc.load_gather(cx_v, [nn])
        py = plsc.load_gather(cy_v, [nn])
        pz = plsc.load_gather(cz_v, [nn])
        nx = plsc.load_gather(nx_v, [nn])
        ny = plsc.load_gather(ny_v, [nn])
        nz = plsc.load_gather(nz_v, [nn])
        d = (qx - px) * nx
        d = d + (qy - py) * ny
        d = d + (qz - pz) * nz
        pen = jnp.maximum(EPS - d, 0.0)
        pen3 = pen * pen * pen
        valid = (base + ridx) < N_CLOTH
        return acc + jnp.where(valid, pen3, 0.0)

    acc = lax.fori_loop(0, Q_GROUPS, body, jnp.zeros((L,), jnp.float32))
    acc_v[...] = acc
    pltpu.sync_copy(acc_v, out_hbm.at[pl.ds(wid * L, L)])


@functools.lru_cache(maxsize=None)
def _sc_pen_kernel():
    return pl.kernel(
        _sc_pen_body,
        out_type=jax.ShapeDtypeStruct((NW * L,), jnp.float32),
        mesh=plsc.VectorSubcoreMesh(core_axis_name="c", subcore_axis_name="s"),
        compiler_params=pltpu.CompilerParams(needs_layout_passes=False),
        scratch_types=[pltpu.VMEM((F_PAD,), jnp.float32)] * 6 + [
            pltpu.VMEM((Q_PER_TILE,), jnp.int32),
            pltpu.VMEM((Q_PER_TILE * 3,), jnp.float32),
            pltpu.VMEM((L,), jnp.float32),
        ],
    )


def kernel(obstacle_pos, obstacle_prev_pos, obstacle_faces, cloth_pos,
           cloth_pred_pos, iter_num):
    faces_p = jnp.pad(obstacle_faces, ((0, F_PAD - N_FACES), (0, 0)))
    cloth_pos_p = jnp.pad(cloth_pos, ((0, Q_PAD - N_CLOTH), (0, 0)))
    pred_p = jnp.pad(cloth_pred_pos, ((0, Q_PAD - N_CLOTH), (0, 0)))

    kx, ky, kz, cx, cy, cz, nx, ny, nz, kh = _sc_prep_kernel()(
        obstacle_pos.reshape(-1), obstacle_prev_pos.reshape(-1),
        faces_p.reshape(-1))
    nn = _tc_knn(cloth_pos_p, kx.reshape(1, F_PAD), ky.reshape(1, F_PAD),
                 kz.reshape(1, F_PAD), kh.reshape(1, F_PAD)).reshape(Q_PAD)
    partial = _sc_pen_kernel()(cx, cy, cz, nx, ny, nz, nn, pred_p.reshape(-1))

    it = jnp.maximum(iter_num - START_RAMPUP, 0)
    progress = jnp.minimum(it / N_RAMPUP, 1.0)
    weight = WEIGHT_START + (WEIGHT_MAX - WEIGHT_START) * progress
    return partial.sum() * weight


# speedup vs baseline: 2.0340x; 2.0340x over previous
import functools

import jax
import jax.numpy as jnp
from jax import lax
from jax.experimental import pallas as pl
from jax.experimental.pallas import tpu as pltpu
from jax.experimental.pallas import tpu_sc as plsc

EPS = 1e-3
WEIGHT_START = 5000.0
WEIGHT_MAX = 500000.0
START_RAMPUP = 50000
N_RAMPUP = 100000

N_OBS = 6890
N_FACES = 13776
N_CLOTH = 20000

F_PAD = 13824
F_PER_TILE = 432
F_GROUPS = F_PER_TILE // 16

Q_PAD = 20480
Q_PER_TILE = 640
Q_GROUPS = Q_PER_TILE // 16

NC, NS, L = 2, 16, 16
NW = NC * NS


def _splat_i32(v):
    return jnp.full((L,), v, dtype=jnp.int32)


def _rsqrt_newton(s):
    bits = plsc.bitcast(s, jnp.int32)
    r = plsc.bitcast(jnp.int32(0x5F3759DF) - (bits >> 1), jnp.float32)
    for _ in range(3):
        r = r * (1.5 - 0.5 * s * r * r)
    return r


def _sc_prep_body(pos_hbm, prev_hbm, faces_hbm, *refs):
    out_hbm = refs[:10]
    pos_v, prev_v, faces_v = refs[10:13]
    buf_v = refs[13:23]
    wid = lax.axis_index("s") * NC + lax.axis_index("c")
    base = wid * F_PER_TILE
    pltpu.sync_copy(pos_hbm, pos_v)
    pltpu.sync_copy(prev_hbm, prev_v)
    pltpu.sync_copy(faces_hbm.at[pl.ds(base * 3, F_PER_TILE * 3)], faces_v)

    iota = lax.iota(jnp.int32, L)

    def body(g, carry):
        ridx = g * L + iota
        fidx = ridx * 3
        i0 = plsc.load_gather(faces_v, [fidx]) * 3
        i1 = plsc.load_gather(faces_v, [fidx + 1]) * 3
        i2 = plsc.load_gather(faces_v, [fidx + 2]) * 3
        valid = (base + ridx) < N_FACES
        cur = []
        keyc = []
        for c in range(3):
            p0 = plsc.load_gather(prev_v, [i0 + c])
            p1 = plsc.load_gather(prev_v, [i1 + c])
            p2 = plsc.load_gather(prev_v, [i2 + c])
            key = (p0 + p1 + p2) / 3.0
            key = jnp.where(valid, key, 1e9)
            keyc.append(key)
            plsc.store_scatter(buf_v[c], [ridx], key)
            v0 = plsc.load_gather(pos_v, [i0 + c])
            v1 = plsc.load_gather(pos_v, [i1 + c])
            v2 = plsc.load_gather(pos_v, [i2 + c])
            plsc.store_scatter(buf_v[3 + c], [ridx], (v0 + v1 + v2) / 3.0)
            cur.append((v0, v1, v2))
        khalf = (keyc[0] * keyc[0] + keyc[1] * keyc[1]
                 + keyc[2] * keyc[2]) * 0.5
        plsc.store_scatter(buf_v[9], [ridx], khalf)
        (v0x, v1x, v2x), (v0y, v1y, v2y), (v0z, v1z, v2z) = cur
        e1x, e1y, e1z = v1x - v0x, v1y - v0y, v1z - v0z
        e2x, e2y, e2z = v2x - v0x, v2y - v0y, v2z - v0z
        crx = e1y * e2z - e1z * e2y
        cry = e1z * e2x - e1x * e2z
        crz = e1x * e2y - e1y * e2x
        s = crx * crx + cry * cry + crz * crz
        r = _rsqrt_newton(s)
        inv = 1.0 / (s * r + 1e-8)
        plsc.store_scatter(buf_v[6], [ridx], crx * inv)
        plsc.store_scatter(buf_v[7], [ridx], cry * inv)
        plsc.store_scatter(buf_v[8], [ridx], crz * inv)
        return carry

    lax.fori_loop(0, F_GROUPS, body, 0)
    for r in range(10):
        pltpu.sync_copy(buf_v[r], out_hbm[r].at[pl.ds(base, F_PER_TILE)])


@functools.lru_cache(maxsize=None)
def _sc_prep_kernel():
    return pl.kernel(
        _sc_prep_body,
        out_type=[jax.ShapeDtypeStruct((F_PAD,), jnp.float32)] * 10,
        mesh=plsc.VectorSubcoreMesh(core_axis_name="c", subcore_axis_name="s"),
        compiler_params=pltpu.CompilerParams(needs_layout_passes=False),
        scratch_types=[
            pltpu.VMEM((N_OBS * 3,), jnp.float32),
            pltpu.VMEM((N_OBS * 3,), jnp.float32),
            pltpu.VMEM((F_PER_TILE * 3,), jnp.int32),
        ] + [pltpu.VMEM((F_PER_TILE,), jnp.float32)] * 10,
    )


def _tc_knn_body(q_ref, kx_ref, ky_ref, kz_ref, kh_ref, out_ref):
    kx = kx_ref[...]
    ky = ky_ref[...]
    kz = kz_ref[...]
    kh = kh_ref[...]
    for j in range(8):
        q = q_ref[pl.ds(j * 128, 128), :]
        t = q[:, 0:1] * kx
        t = t + q[:, 1:2] * ky
        t = t + q[:, 2:3] * kz
        score = kh - t
        smin = jnp.min(score, axis=1, keepdims=True)
        lane = lax.broadcasted_iota(jnp.int32, (128, F_PAD), 1)
        cand = jnp.where(score == smin, lane, 1 << 30)
        out_ref[0, j, :] = jnp.min(cand, axis=1)


_tc_knn = pl.pallas_call(
    _tc_knn_body,
    grid=(Q_PAD // 1024,),
    in_specs=[
        pl.BlockSpec((1024, 3), lambda i: (i, 0)),
        pl.BlockSpec((1, F_PAD), lambda i: (0, 0)),
        pl.BlockSpec((1, F_PAD), lambda i: (0, 0)),
        pl.BlockSpec((1, F_PAD), lambda i: (0, 0)),
        pl.BlockSpec((1, F_PAD), lambda i: (0, 0)),
    ],
    out_specs=pl.BlockSpec((1, 8, 128), lambda i: (i, 0, 0)),
    out_shape=jax.ShapeDtypeStruct((Q_PAD // 1024, 8, 128), jnp.int32),
)


def _sc_pen_body(cx_hbm, cy_hbm, cz_hbm, nx_hbm, ny_hbm, nz_hbm,
                 nn_hbm, pred_hbm, out_hbm,
                 cx_v, cy_v, cz_v, nx_v, ny_v, nz_v, nn_v, pred_v, acc_v):
    wid = lax.axis_index("s") * NC + lax.axis_index("c")
    base = wid * Q_PER_TILE
    for h, v in ((cx_hbm, cx_v), (cy_hbm, cy_v), (cz_hbm, cz_v),
                 (nx_hbm, nx_v), (ny_hbm, ny_v), (nz_hbm, nz_v)):
        pltpu.sync_copy(h, v)
    pltpu.sync_copy(nn_hbm.at[pl.ds(base, Q_PER_TILE)], nn_v)
    pltpu.sync_copy(pred_hbm.at[pl.ds(base * 3, Q_PER_TILE * 3)], pred_v)

    iota = lax.iota(jnp.int32, L)

    def body(g, acc):
        ridx = g * L + iota
        qidx = ridx * 3
        nn = plsc.load_gather(nn_v, [ridx])
        qx = plsc.load_gather(pred_v, [qidx])
        qy = plsc.load_gather(pred_v, [qidx + 1])
        qz = plsc.load_gather(pred_v, [qidx + 2])
        px = plsc.load_gather(cx_v, [nn])
        py = plsc.load_gather(cy_v, [nn])
        pz = plsc.load_gather(cz_v, [nn])
        nx = plsc.load_gather(nx_v, [nn])
        ny = plsc.load_gather(ny_v, [nn])
        nz = plsc.load_gather(nz_v, [nn])
        d = (qx - px) * nx
        d = d + (qy - py) * ny
        d = d + (qz - pz) * nz
        pen = jnp.maximum(EPS - d, 0.0)
        pen3 = pen * pen * pen
        valid = (base + ridx) < N_CLOTH
        return acc + jnp.where(valid, pen3, 0.0)

    acc = lax.fori_loop(0, Q_GROUPS, body, jnp.zeros((L,), jnp.float32))
    acc_v[...] = acc
    pltpu.sync_copy(acc_v, out_hbm.at[pl.ds(wid * L, L)])


@functools.lru_cache(maxsize=None)
def _sc_pen_kernel():
    return pl.kernel(
        _sc_pen_body,
        out_type=jax.ShapeDtypeStruct((NW * L,), jnp.float32),
        mesh=plsc.VectorSubcoreMesh(core_axis_name="c", subcore_axis_name="s"),
        compiler_params=pltpu.CompilerParams(needs_layout_passes=False),
        scratch_types=[pltpu.VMEM((F_PAD,), jnp.float32)] * 6 + [
            pltpu.VMEM((Q_PER_TILE,), jnp.int32),
            pltpu.VMEM((Q_PER_TILE * 3,), jnp.float32),
            pltpu.VMEM((L,), jnp.float32),
        ],
    )


def kernel(obstacle_pos, obstacle_prev_pos, obstacle_faces, cloth_pos,
           cloth_pred_pos, iter_num):
    faces_p = jnp.pad(obstacle_faces, ((0, F_PAD - N_FACES), (0, 0)))
    cloth_pos_p = jnp.pad(cloth_pos, ((0, Q_PAD - N_CLOTH), (0, 0)))
    pred_p = jnp.pad(cloth_pred_pos, ((0, Q_PAD - N_CLOTH), (0, 0)))

    kx, ky, kz, cx, cy, cz, nx, ny, nz, kh = _sc_prep_kernel()(
        obstacle_pos.reshape(-1), obstacle_prev_pos.reshape(-1),
        faces_p.reshape(-1))
    nn = _tc_knn(cloth_pos_p, kx.reshape(1, F_PAD), ky.reshape(1, F_PAD),
                 kz.reshape(1, F_PAD), kh.reshape(1, F_PAD)).reshape(Q_PAD)
    partial = _sc_pen_kernel()(cx, cy, cz, nx, ny, nz, nn, pred_p.reshape(-1))

    it = jnp.maximum(iter_num - START_RAMPUP, 0)
    progress = jnp.minimum(it / N_RAMPUP, 1.0)
    weight = WEIGHT_START + (WEIGHT_MAX - WEIGHT_START) * progress
    return partial.sum() * weight


# speedup vs baseline: 2.3844x; 1.1723x over previous
import functools

import jax
import jax.numpy as jnp
from jax import lax
from jax.experimental import pallas as pl
from jax.experimental.pallas import tpu as pltpu
from jax.experimental.pallas import tpu_sc as plsc

EPS = 1e-3
WEIGHT_START = 5000.0
WEIGHT_MAX = 500000.0
START_RAMPUP = 50000
N_RAMPUP = 100000

N_OBS = 6890
N_FACES = 13776
N_CLOTH = 20000

F_PAD = 13824
F_PER_TILE = 432
F_GROUPS = F_PER_TILE // 16

Q_PAD = 20480
Q_PER_TILE = 640
Q_GROUPS = Q_PER_TILE // 16

NC, NS, L = 2, 16, 16
NW = NC * NS


def _splat_i32(v):
    return jnp.full((L,), v, dtype=jnp.int32)


def _rsqrt_newton(s):
    bits = plsc.bitcast(s, jnp.int32)
    r = plsc.bitcast(jnp.int32(0x5F3759DF) - (bits >> 1), jnp.float32)
    for _ in range(3):
        r = r * (1.5 - 0.5 * s * r * r)
    return r


def _sc_prep_body(pos_hbm, prev_hbm, faces_hbm, *refs):
    out_hbm = refs[:10]
    pos_v, prev_v, faces_v = refs[10:13]
    buf_v = refs[13:23]
    wid = lax.axis_index("s") * NC + lax.axis_index("c")
    base = wid * F_PER_TILE
    pltpu.sync_copy(pos_hbm, pos_v)
    pltpu.sync_copy(prev_hbm, prev_v)
    pltpu.sync_copy(faces_hbm.at[pl.ds(base * 3, F_PER_TILE * 3)], faces_v)

    iota = lax.iota(jnp.int32, L)

    def body(g, carry):
        ridx = g * L + iota
        fidx = ridx * 3
        i0 = plsc.load_gather(faces_v, [fidx]) * 3
        i1 = plsc.load_gather(faces_v, [fidx + 1]) * 3
        i2 = plsc.load_gather(faces_v, [fidx + 2]) * 3
        valid = (base + ridx) < N_FACES
        cur = []
        keyc = []
        for c in range(3):
            p0 = plsc.load_gather(prev_v, [i0 + c])
            p1 = plsc.load_gather(prev_v, [i1 + c])
            p2 = plsc.load_gather(prev_v, [i2 + c])
            key = (p0 + p1 + p2) / 3.0
            key = jnp.where(valid, key, 1e9)
            keyc.append(key)
            plsc.store_scatter(buf_v[c], [ridx], key)
            v0 = plsc.load_gather(pos_v, [i0 + c])
            v1 = plsc.load_gather(pos_v, [i1 + c])
            v2 = plsc.load_gather(pos_v, [i2 + c])
            plsc.store_scatter(buf_v[3 + c], [ridx], (v0 + v1 + v2) / 3.0)
            cur.append((v0, v1, v2))
        khalf = (keyc[0] * keyc[0] + keyc[1] * keyc[1]
                 + keyc[2] * keyc[2]) * 0.5
        plsc.store_scatter(buf_v[9], [ridx], khalf)
        (v0x, v1x, v2x), (v0y, v1y, v2y), (v0z, v1z, v2z) = cur
        e1x, e1y, e1z = v1x - v0x, v1y - v0y, v1z - v0z
        e2x, e2y, e2z = v2x - v0x, v2y - v0y, v2z - v0z
        crx = e1y * e2z - e1z * e2y
        cry = e1z * e2x - e1x * e2z
        crz = e1x * e2y - e1y * e2x
        s = crx * crx + cry * cry + crz * crz
        r = _rsqrt_newton(s)
        inv = 1.0 / (s * r + 1e-8)
        plsc.store_scatter(buf_v[6], [ridx], crx * inv)
        plsc.store_scatter(buf_v[7], [ridx], cry * inv)
        plsc.store_scatter(buf_v[8], [ridx], crz * inv)
        return carry

    lax.fori_loop(0, F_GROUPS, body, 0)
    for r in range(10):
        pltpu.sync_copy(buf_v[r], out_hbm[r].at[pl.ds(base, F_PER_TILE)])


@functools.lru_cache(maxsize=None)
def _sc_prep_kernel():
    return pl.kernel(
        _sc_prep_body,
        out_type=[jax.ShapeDtypeStruct((F_PAD,), jnp.float32)] * 10,
        mesh=plsc.VectorSubcoreMesh(core_axis_name="c", subcore_axis_name="s"),
        compiler_params=pltpu.CompilerParams(needs_layout_passes=False),
        scratch_types=[
            pltpu.VMEM((N_OBS * 3,), jnp.float32),
            pltpu.VMEM((N_OBS * 3,), jnp.float32),
            pltpu.VMEM((F_PER_TILE * 3,), jnp.int32),
        ] + [pltpu.VMEM((F_PER_TILE,), jnp.float32)] * 10,
    )


def _tc_knn_body(q_ref, kx_ref, ky_ref, kz_ref, kh_ref, out_ref):
    for j in range(8):
        q = q_ref[pl.ds(j * 128, 128), :]
        qx = q[:, 0:1]
        qy = q[:, 1:2]
        qz = q[:, 2:3]
        m = jnp.full((128, 128), 3.4e38, dtype=jnp.float32)
        idx = jnp.zeros((128, 128), dtype=jnp.int32)
        for kc in range(F_PAD // 128):
            lo = kc * 128
            t = qx * kx_ref[0:1, lo:lo + 128]
            t = t + qy * ky_ref[0:1, lo:lo + 128]
            t = t + qz * kz_ref[0:1, lo:lo + 128]
            s = kh_ref[0:1, lo:lo + 128] - t
            idx = jnp.where(s < m, kc, idx)
            m = jnp.minimum(m, s)
        lane = lax.broadcasted_iota(jnp.int32, (128, 128), 1)
        glob = idx * 128 + lane
        smin = jnp.min(m, axis=1, keepdims=True)
        cand = jnp.where(m == smin, glob, 1 << 30)
        out_ref[0, j, :] = jnp.min(cand, axis=1)


_tc_knn = pl.pallas_call(
    _tc_knn_body,
    grid=(Q_PAD // 1024,),
    in_specs=[
        pl.BlockSpec((1024, 3), lambda i: (i, 0)),
        pl.BlockSpec((1, F_PAD), lambda i: (0, 0)),
        pl.BlockSpec((1, F_PAD), lambda i: (0, 0)),
        pl.BlockSpec((1, F_PAD), lambda i: (0, 0)),
        pl.BlockSpec((1, F_PAD), lambda i: (0, 0)),
    ],
    out_specs=pl.BlockSpec((1, 8, 128), lambda i: (i, 0, 0)),
    out_shape=jax.ShapeDtypeStruct((Q_PAD // 1024, 8, 128), jnp.int32),
)


def _sc_pen_body(cx_hbm, cy_hbm, cz_hbm, nx_hbm, ny_hbm, nz_hbm,
                 nn_hbm, pred_hbm, out_hbm,
                 cx_v, cy_v, cz_v, nx_v, ny_v, nz_v, nn_v, pred_v, acc_v):
    wid = lax.axis_index("s") * NC + lax.axis_index("c")
    base = wid * Q_PER_TILE
    for h, v in ((cx_hbm, cx_v), (cy_hbm, cy_v), (cz_hbm, cz_v),
                 (nx_hbm, nx_v), (ny_hbm, ny_v), (nz_hbm, nz_v)):
        pltpu.sync_copy(h, v)
    pltpu.sync_copy(nn_hbm.at[pl.ds(base, Q_PER_TILE)], nn_v)
    pltpu.sync_copy(pred_hbm.at[pl.ds(base * 3, Q_PER_TILE * 3)], pred_v)

    iota = lax.iota(jnp.int32, L)

    def body(g, acc):
        ridx = g * L + iota
        qidx = ridx * 3
        nn = plsc.load_gather(nn_v, [ridx])
        qx = plsc.load_gather(pred_v, [qidx])
        qy = plsc.load_gather(pred_v, [qidx + 1])
        qz = plsc.load_gather(pred_v, [qidx + 2])
        px = plsc.load_gather(cx_v, [nn])
        py = plsc.load_gather(cy_v, [nn])
        pz = plsc.load_gather(cz_v, [nn])
        nx = plsc.load_gather(nx_v, [nn])
        ny = plsc.load_gather(ny_v, [nn])
        nz = plsc.load_gather(nz_v, [nn])
        d = (qx - px) * nx
        d = d + (qy - py) * ny
        d = d + (qz - pz) * nz
        pen = jnp.maximum(EPS - d, 0.0)
        pen3 = pen * pen * pen
        valid = (base + ridx) < N_CLOTH
        return acc + jnp.where(valid, pen3, 0.0)

    acc = lax.fori_loop(0, Q_GROUPS, body, jnp.zeros((L,), jnp.float32))
    acc_v[...] = acc
    pltpu.sync_copy(acc_v, out_hbm.at[pl.ds(wid * L, L)])


@functools.lru_cache(maxsize=None)
def _sc_pen_kernel():
    return pl.kernel(
        _sc_pen_body,
        out_type=jax.ShapeDtypeStruct((NW * L,), jnp.float32),
        mesh=plsc.VectorSubcoreMesh(core_axis_name="c", subcore_axis_name="s"),
        compiler_params=pltpu.CompilerParams(needs_layout_passes=False),
        scratch_types=[pltpu.VMEM((F_PAD,), jnp.float32)] * 6 + [
            pltpu.VMEM((Q_PER_TILE,), jnp.int32),
            pltpu.VMEM((Q_PER_TILE * 3,), jnp.float32),
            pltpu.VMEM((L,), jnp.float32),
        ],
    )


def kernel(obstacle_pos, obstacle_prev_pos, obstacle_faces, cloth_pos,
           cloth_pred_pos, iter_num):
    faces_p = jnp.pad(obstacle_faces, ((0, F_PAD - N_FACES), (0, 0)))
    cloth_pos_p = jnp.pad(cloth_pos, ((0, Q_PAD - N_CLOTH), (0, 0)))
    pred_p = jnp.pad(cloth_pred_pos, ((0, Q_PAD - N_CLOTH), (0, 0)))

    kx, ky, kz, cx, cy, cz, nx, ny, nz, kh = _sc_prep_kernel()(
        obstacle_pos.reshape(-1), obstacle_prev_pos.reshape(-1),
        faces_p.reshape(-1))
    nn = _tc_knn(cloth_pos_p, kx.reshape(1, F_PAD), ky.reshape(1, F_PAD),
                 kz.reshape(1, F_PAD), kh.reshape(1, F_PAD)).reshape(Q_PAD)
    partial = _sc_pen_kernel()(cx, cy, cz, nx, ny, nz, nn, pred_p.reshape(-1))

    it = jnp.maximum(iter_num - START_RAMPUP, 0)
    progress = jnp.minimum(it / N_RAMPUP, 1.0)
    weight = WEIGHT_START + (WEIGHT_MAX - WEIGHT_START) * progress
    return partial.sum() * weight
